# Initial kernel scaffold; baseline (speedup 1.0000x reference)
import functools

import jax
import jax.numpy as jnp
from jax import lax
from jax.experimental import pallas as pl
from jax.experimental.pallas import tpu as pltpu
from jax.experimental.pallas import tpu_sc as plsc

B, S, N = 8, 512, 2048
LOW_C, HIGH_C = 256, 128
MID = HIGH_C // 4
POS_H = 64
ATTN_M = 4
K_NEI = 16
NBLK = 256
NROWB = N // NBLK

_INF = jnp.float32(jnp.inf)


def _interp_body(plow_ref, upw_ref, upb_ref, xh_ref, xl_ref, ph_ref, out_ref):
    pl_feat = jnp.maximum(
        jnp.dot(plow_ref[...], upw_ref[...], preferred_element_type=jnp.float32)
        + upb_ref[...], 0.0)
    xh = xh_ref[...]
    xl = xl_ref[...]
    rn = jnp.sum(xh * xh, axis=1, keepdims=True)
    cn = jnp.sum(xl * xl, axis=0, keepdims=True)
    d2 = rn + cn - 2.0 * jnp.dot(xh, xl, preferred_element_type=jnp.float32)
    lane = lax.broadcasted_iota(jnp.int32, (N, S), 1)
    w_mat = jnp.zeros((N, S), jnp.float32)
    recips = []
    onehots = []
    for _ in range(3):
        m = jnp.min(d2, axis=1, keepdims=True)
        idx = jnp.min(jnp.where(d2 == m, lane, S), axis=1, keepdims=True)
        d2 = jnp.where(lane == idx, _INF, d2)
        recips.append(1.0 / (m + 1e-8))
        onehots.append(lane == idx)
    norm = recips[0] + recips[1] + recips[2]
    for r, oh in zip(recips, onehots):
        w_mat = w_mat + jnp.where(oh, r / norm, 0.0)
    out_ref[...] = (
        jnp.dot(w_mat, pl_feat, preferred_element_type=jnp.float32)
        + ph_ref[...])


def _interp(plowT, upwT, upb, xhT, xl, phT):
    return pl.pallas_call(
        _interp_body,
        grid=(B,),
        in_specs=[
            pl.BlockSpec((None, S, LOW_C), lambda b: (b, 0, 0)),
            pl.BlockSpec((LOW_C, HIGH_C), lambda b: (0, 0)),
            pl.BlockSpec((1, HIGH_C), lambda b: (0, 0)),
            pl.BlockSpec((None, N, 3), lambda b: (b, 0, 0)),
            pl.BlockSpec((None, 3, S), lambda b: (b, 0, 0)),
            pl.BlockSpec((None, N, HIGH_C), lambda b: (b, 0, 0)),
        ],
        out_specs=pl.BlockSpec((None, N, HIGH_C), lambda b: (b, 0, 0)),
        out_shape=jax.ShapeDtypeStruct((B, N, HIGH_C), jnp.float32),
    )(plowT, upwT, upb, xhT, xl, phT)


def _knn_body(xb_ref, xf_ref, out_ref):
    b = pl.program_id(0)
    xb = xb_ref[...]
    xf = xf_ref[...]
    rn = jnp.sum(xb * xb, axis=1, keepdims=True)
    cn = jnp.sum(xf * xf, axis=0, keepdims=True)
    d2 = rn + cn - 2.0 * jnp.dot(xb, xf, preferred_element_type=jnp.float32)
    lane = lax.broadcasted_iota(jnp.int32, (NBLK, N), 1)
    bits = lax.bitcast_convert_type(d2, jnp.int32)
    packed = lax.bitcast_convert_type((bits & ~2047) | lane, jnp.float32)
    cols = []
    for _ in range(K_NEI):
        m = jnp.min(packed, axis=1, keepdims=True)
        packed = jnp.where(packed == m, _INF, packed)
        cols.append(lax.bitcast_convert_type(m, jnp.int32) & 2047)
    idx = jnp.concatenate(cols, axis=1)
    out_ref[...] = idx + b * N


def _knn(xhT, xf):
    return pl.pallas_call(
        _knn_body,
        grid=(B, NROWB),
        in_specs=[
            pl.BlockSpec((None, NBLK, 3), lambda b, r: (b, r, 0)),
            pl.BlockSpec((None, 3, N), lambda b, r: (b, 0, 0)),
        ],
        out_specs=pl.BlockSpec((None, NBLK, K_NEI), lambda b, r: (b, r, 0)),
        out_shape=jax.ShapeDtypeStruct((B, N, K_NEI), jnp.int32),
    )(xhT, xf)


def _qkv_body(pts_ref, bw_ref, bb_ref, qkvw_ref, q_ref, kv_ref):
    x = (jnp.dot(pts_ref[...], bw_ref[...], preferred_element_type=jnp.float32)
         + bb_ref[...])
    qkv = jnp.dot(x, qkvw_ref[...], preferred_element_type=jnp.float32)
    q_ref[...] = qkv[:, :MID]
    kv_ref[...] = qkv[:, MID:]


def _qkv(pts, beforeT, beforeb, qkvT):
    return pl.pallas_call(
        _qkv_body,
        grid=(B,),
        in_specs=[
            pl.BlockSpec((None, N, HIGH_C), lambda b: (b, 0, 0)),
            pl.BlockSpec((HIGH_C, MID), lambda b: (0, 0)),
            pl.BlockSpec((1, MID), lambda b: (0, 0)),
            pl.BlockSpec((MID, 3 * MID), lambda b: (0, 0)),
        ],
        out_specs=[
            pl.BlockSpec((None, N, MID), lambda b: (b, 0, 0)),
            pl.BlockSpec((None, N, 2 * MID), lambda b: (b, 0, 0)),
        ],
        out_shape=[
            jax.ShapeDtypeStruct((B, N, MID), jnp.float32),
            jax.ShapeDtypeStruct((B, N, 2 * MID), jnp.float32),
        ],
    )(pts, beforeT, beforeb, qkvT)


def _sc_gather(table, idx, d):
    m_tot = idx.shape[0]
    info = plsc.get_sparse_core_info()
    nw = info.num_cores * info.num_subcores
    per_w = m_tot // nw
    ch = 128
    n_ch = per_w // ch
    mesh = plsc.VectorSubcoreMesh(core_axis_name="c", subcore_axis_name="s")

    @functools.partial(
        pl.kernel, mesh=mesh,
        out_type=jax.ShapeDtypeStruct((m_tot, d), jnp.float32),
        scratch_types=[
            pltpu.VMEM((ch,), jnp.int32),
            pltpu.VMEM((ch, d), jnp.float32),
            pltpu.SemaphoreType.DMA,
        ],
    )
    def k(table_hbm, idx_hbm, out_hbm, idx_v, rows_v, sem):
        wid = lax.axis_index("s") * info.num_cores + lax.axis_index("c")
        base = wid * per_w

        def body(i, carry):
            off = base + i * ch
            pltpu.sync_copy(idx_hbm.at[pl.ds(off, ch)], idx_v)
            pltpu.async_copy(table_hbm.at[idx_v], rows_v, sem).wait()
            pltpu.sync_copy(rows_v, out_hbm.at[pl.ds(off, ch)])
            return carry

        lax.fori_loop(0, n_ch, body, 0)

    return k(table, idx)


def _attn_body(pts_ref, q_ref, kvg_ref, xyzg_ref, xhp_ref,
               pw1_ref, pb1_ref, pw2_ref, pb2_ref,
               aw1_ref, ab1_ref, aw2_ref, ab2_ref,
               afw_ref, afb_ref, out_ref):
    rows = NBLK * K_NEI
    kvg = kvg_ref[...]
    kg = kvg[:, :MID]
    vg = kvg[:, MID:]
    xh_rep = jnp.broadcast_to(
        xhp_ref[...][:, None, :], (NBLK, K_NEI, 16)).reshape(rows, 16)
    q_rep = jnp.broadcast_to(
        q_ref[...][:, None, :], (NBLK, K_NEI, MID)).reshape(rows, MID)
    dxyz = xh_rep - xyzg_ref[...]
    h = jnp.maximum(
        jnp.dot(dxyz, pw1_ref[...], preferred_element_type=jnp.float32)
        + pb1_ref[...], 0.0)
    rel = (jnp.dot(h, pw2_ref[...], preferred_element_type=jnp.float32)
           + pb2_ref[...])
    sim_in = q_rep - kg + rel
    hid = jnp.maximum(
        jnp.dot(sim_in, aw1_ref[...], preferred_element_type=jnp.float32)
        + ab1_ref[...], 0.0)
    sim = (jnp.dot(hid, aw2_ref[...], preferred_element_type=jnp.float32)
           + ab2_ref[...])
    e = jnp.exp(sim)
    vr = vg + rel
    num = jnp.sum((e * vr).reshape(NBLK, K_NEI, MID), axis=1)
    den = jnp.sum(e.reshape(NBLK, K_NEI, MID), axis=1)
    agg = num / den
    out_ref[...] = (
        pts_ref[...]
        + jnp.dot(agg, afw_ref[...], preferred_element_type=jnp.float32)
        + afb_ref[...])


def _attn(pts, q, kvg, xyzg, xhp, w):
    rows = NBLK * K_NEI

    def full(shp):
        return pl.BlockSpec(shp, lambda b, r: tuple(0 for _ in shp))

    return pl.pallas_call(
        _attn_body,
        grid=(B, NROWB),
        in_specs=[
            pl.BlockSpec((None, NBLK, HIGH_C), lambda b, r: (b, r, 0)),
            pl.BlockSpec((None, NBLK, MID), lambda b, r: (b, r, 0)),
            pl.BlockSpec((rows, 2 * MID), lambda b, r: (b * NROWB + r, 0)),
            pl.BlockSpec((rows, 16), lambda b, r: (b * NROWB + r, 0)),
            pl.BlockSpec((None, NBLK, 16), lambda b, r: (b, r, 0)),
            full((16, POS_H)), full((1, POS_H)),
            full((POS_H, MID)), full((1, MID)),
            full((MID, MID * ATTN_M)), full((1, MID * ATTN_M)),
            full((MID * ATTN_M, MID)), full((1, MID)),
            full((MID, HIGH_C)), full((1, HIGH_C)),
        ],
        out_specs=pl.BlockSpec((None, NBLK, HIGH_C), lambda b, r: (b, r, 0)),
        out_shape=jax.ShapeDtypeStruct((B, N, HIGH_C), jnp.float32),
    )(pts, q, kvg, xyzg, xhp,
      w['pw1'], w['pb1'], w['pw2'], w['pb2'],
      w['aw1'], w['ab1'], w['aw2'], w['ab2'], w['afw'], w['afb'])


def kernel(xyz_low, xyz_high, points_low, points_high, params):
    f32 = jnp.float32
    xhT = jnp.transpose(xyz_high, (0, 2, 1))
    plowT = jnp.transpose(points_low, (0, 2, 1))
    phT = jnp.transpose(points_high, (0, 2, 1))

    inv = 1.0 / jnp.sqrt(f32(1.0 + 1e-5))
    scale = inv * params['up_bn_g']
    upwT = (params['up_w'] * scale[:, None]).T
    upb = (params['up_b'] * scale + params['up_bn_b']).reshape(1, HIGH_C)

    points = _interp(plowT, upwT, upb, xhT, xyz_low, phT)

    gidx = _knn(xhT, xyz_high)
    idx_flat = gidx.reshape(B * N * K_NEI)

    xhp = jnp.pad(xhT, ((0, 0), (0, 0), (0, 13)))
    xyz_table = xhp.reshape(B * N, 16)
    xyzg = _sc_gather(xyz_table, idx_flat, 16)

    for blk in params['blocks']:
        w = {
            'pw1': jnp.pad(blk['pos_w1'], ((0, 0), (0, 13))).T,
            'pb1': blk['pos_b1'].reshape(1, POS_H),
            'pw2': blk['pos_w2'].T, 'pb2': blk['pos_b2'].reshape(1, MID),
            'aw1': blk['attn_w1'].T, 'ab1': blk['attn_b1'].reshape(1, MID * ATTN_M),
            'aw2': blk['attn_w2'].T, 'ab2': blk['attn_b2'].reshape(1, MID),
            'afw': blk['after_w'].T, 'afb': blk['after_b'].reshape(1, HIGH_C),
        }
        q, kv = _qkv(points, blk['before_w'].T,
                     blk['before_b'].reshape(1, MID), blk['qkv_w'].T)
        kvg = _sc_gather(kv.reshape(B * N, 2 * MID), idx_flat, 2 * MID)
        points = _attn(points, q, kvg, xyzg, xhp, w)

    return jnp.transpose(points, (0, 2, 1))


# speedup vs baseline: 25.5676x; 25.5676x over previous
import functools

import jax
import jax.numpy as jnp
from jax import lax
from jax.experimental import pallas as pl
from jax.experimental.pallas import tpu as pltpu
from jax.experimental.pallas import tpu_sc as plsc

B, S, N = 8, 512, 2048
LOW_C, HIGH_C = 256, 128
MID = HIGH_C // 4
POS_H = 64
ATTN_M = 4
K_NEI = 16
NBLK = 256
NROWB = N // NBLK

_INF = float('inf')


def _interp_body(plow_ref, upw_ref, upb_ref, xh_ref, xl_ref, ph_ref, out_ref):
    pl_feat = jnp.maximum(
        jnp.dot(plow_ref[...], upw_ref[...], preferred_element_type=jnp.float32)
        + upb_ref[...], 0.0)
    xh = xh_ref[...]
    xl = xl_ref[...]
    rn = jnp.sum(xh * xh, axis=1, keepdims=True)
    cn = jnp.sum(xl * xl, axis=0, keepdims=True)
    d2 = rn + cn - 2.0 * jnp.dot(xh, xl, preferred_element_type=jnp.float32)
    lane = lax.broadcasted_iota(jnp.int32, (N, S), 1)
    w_mat = jnp.zeros((N, S), jnp.float32)
    recips = []
    onehots = []
    for _ in range(3):
        m = jnp.min(d2, axis=1, keepdims=True)
        idx = jnp.min(jnp.where(d2 == m, lane, S), axis=1, keepdims=True)
        d2 = jnp.where(lane == idx, _INF, d2)
        recips.append(1.0 / (m + 1e-8))
        onehots.append(lane == idx)
    norm = recips[0] + recips[1] + recips[2]
    for r, oh in zip(recips, onehots):
        w_mat = w_mat + jnp.where(oh, r / norm, 0.0)
    out_ref[...] = (
        jnp.dot(w_mat, pl_feat, preferred_element_type=jnp.float32)
        + ph_ref[...])


def _interp(plowT, upwT, upb, xhT, xl, phT):
    return pl.pallas_call(
        _interp_body,
        grid=(B,),
        in_specs=[
            pl.BlockSpec((None, S, LOW_C), lambda b: (b, 0, 0)),
            pl.BlockSpec((LOW_C, HIGH_C), lambda b: (0, 0)),
            pl.BlockSpec((1, HIGH_C), lambda b: (0, 0)),
            pl.BlockSpec((None, N, 3), lambda b: (b, 0, 0)),
            pl.BlockSpec((None, 3, S), lambda b: (b, 0, 0)),
            pl.BlockSpec((None, N, HIGH_C), lambda b: (b, 0, 0)),
        ],
        out_specs=pl.BlockSpec((None, N, HIGH_C), lambda b: (b, 0, 0)),
        out_shape=jax.ShapeDtypeStruct((B, N, HIGH_C), jnp.float32),
    )(plowT, upwT, upb, xhT, xl, phT)


def _knn_body(xb_ref, xf_ref, out_ref):
    b = pl.program_id(0)
    xb = xb_ref[...]
    xf = xf_ref[...]
    rn = jnp.sum(xb * xb, axis=1, keepdims=True)
    cn = jnp.sum(xf * xf, axis=0, keepdims=True)
    d2 = rn + cn - 2.0 * jnp.dot(xb, xf, preferred_element_type=jnp.float32)
    lane = lax.broadcasted_iota(jnp.int32, (NBLK, N), 1)
    bits = lax.bitcast_convert_type(d2, jnp.int32)
    packed = lax.bitcast_convert_type((bits & ~2047) | lane, jnp.float32)
    cols = []
    for _ in range(K_NEI):
        m = jnp.min(packed, axis=1, keepdims=True)
        packed = jnp.where(packed == m, _INF, packed)
        cols.append(lax.bitcast_convert_type(m, jnp.int32) & 2047)
    idx = jnp.concatenate(cols, axis=1)
    out_ref[...] = idx + b * N


def _knn(xhT, xf):
    return pl.pallas_call(
        _knn_body,
        grid=(B, NROWB),
        in_specs=[
            pl.BlockSpec((None, NBLK, 3), lambda b, r: (b, r, 0)),
            pl.BlockSpec((None, 3, N), lambda b, r: (b, 0, 0)),
        ],
        out_specs=pl.BlockSpec((None, NBLK, K_NEI), lambda b, r: (b, r, 0)),
        out_shape=jax.ShapeDtypeStruct((B, N, K_NEI), jnp.int32),
    )(xhT, xf)


def _qkv_body(pts_ref, bw_ref, bb_ref, qkvw_ref, q_ref, kv_ref):
    x = (jnp.dot(pts_ref[...], bw_ref[...], preferred_element_type=jnp.float32)
         + bb_ref[...])
    qkv = jnp.dot(x, qkvw_ref[...], preferred_element_type=jnp.float32)
    q_ref[...] = qkv[:, :MID]
    kv_ref[...] = qkv[:, MID:]


def _qkv(pts, beforeT, beforeb, qkvT):
    return pl.pallas_call(
        _qkv_body,
        grid=(B,),
        in_specs=[
            pl.BlockSpec((None, N, HIGH_C), lambda b: (b, 0, 0)),
            pl.BlockSpec((HIGH_C, MID), lambda b: (0, 0)),
            pl.BlockSpec((1, MID), lambda b: (0, 0)),
            pl.BlockSpec((MID, 3 * MID), lambda b: (0, 0)),
        ],
        out_specs=[
            pl.BlockSpec((None, N, MID), lambda b: (b, 0, 0)),
            pl.BlockSpec((None, N, 2 * MID), lambda b: (b, 0, 0)),
        ],
        out_shape=[
            jax.ShapeDtypeStruct((B, N, MID), jnp.float32),
            jax.ShapeDtypeStruct((B, N, 2 * MID), jnp.float32),
        ],
    )(pts, beforeT, beforeb, qkvT)


def _sc_gather(table, idx, d):
    m_tot = idx.shape[0]
    info = plsc.get_sparse_core_info()
    nw = info.num_cores * info.num_subcores
    per_w = m_tot // nw
    ch = 128
    n_ch = per_w // ch
    mesh = plsc.VectorSubcoreMesh(core_axis_name="c", subcore_axis_name="s")

    @functools.partial(
        pl.kernel, mesh=mesh,
        compiler_params=pltpu.CompilerParams(use_tc_tiling_on_sc=False),
        out_type=jax.ShapeDtypeStruct((m_tot, d), jnp.float32),
        scratch_types=[
            pltpu.VMEM((ch,), jnp.int32),
            pltpu.VMEM((ch, d), jnp.float32),
            pltpu.SemaphoreType.DMA,
        ],
    )
    def k(table_hbm, idx_hbm, out_hbm, idx_v, rows_v, sem):
        wid = lax.axis_index("s") * info.num_cores + lax.axis_index("c")
        base = wid * per_w

        def body(i, carry):
            off = base + i * ch
            pltpu.sync_copy(idx_hbm.at[pl.ds(off, ch)], idx_v)
            pltpu.async_copy(table_hbm.at[idx_v], rows_v, sem).wait()
            pltpu.sync_copy(rows_v, out_hbm.at[pl.ds(off, ch)])
            return carry

        lax.fori_loop(0, n_ch, body, 0)

    return k(table, idx)


def _attn_body(pts_ref, q_ref, kvg_ref, xyzg_ref, xhp_ref,
               pw1_ref, pb1_ref, pw2_ref, pb2_ref,
               aw1_ref, ab1_ref, aw2_ref, ab2_ref,
               afw_ref, afb_ref, out_ref):
    rows = NBLK * K_NEI
    kvg = kvg_ref[...]
    kg = kvg[:, :MID]
    vg = kvg[:, MID:]
    xh_rep = jnp.broadcast_to(
        xhp_ref[...][:, None, :], (NBLK, K_NEI, 16)).reshape(rows, 16)
    q_rep = jnp.broadcast_to(
        q_ref[...][:, None, :], (NBLK, K_NEI, MID)).reshape(rows, MID)
    dxyz = xh_rep - xyzg_ref[...]
    h = jnp.maximum(
        jnp.dot(dxyz, pw1_ref[...], preferred_element_type=jnp.float32)
        + pb1_ref[...], 0.0)
    rel = (jnp.dot(h, pw2_ref[...], preferred_element_type=jnp.float32)
           + pb2_ref[...])
    sim_in = q_rep - kg + rel
    hid = jnp.maximum(
        jnp.dot(sim_in, aw1_ref[...], preferred_element_type=jnp.float32)
        + ab1_ref[...], 0.0)
    sim = (jnp.dot(hid, aw2_ref[...], preferred_element_type=jnp.float32)
           + ab2_ref[...])
    e = jnp.exp(sim)
    vr = vg + rel
    num = jnp.sum((e * vr).reshape(NBLK, K_NEI, MID), axis=1)
    den = jnp.sum(e.reshape(NBLK, K_NEI, MID), axis=1)
    agg = num / den
    out_ref[...] = (
        pts_ref[...]
        + jnp.dot(agg, afw_ref[...], preferred_element_type=jnp.float32)
        + afb_ref[...])


def _attn(pts, q, kvg, xyzg, xhp, w):
    rows = NBLK * K_NEI

    def full(shp):
        return pl.BlockSpec(shp, lambda b, r: tuple(0 for _ in shp))

    return pl.pallas_call(
        _attn_body,
        grid=(B, NROWB),
        in_specs=[
            pl.BlockSpec((None, NBLK, HIGH_C), lambda b, r: (b, r, 0)),
            pl.BlockSpec((None, NBLK, MID), lambda b, r: (b, r, 0)),
            pl.BlockSpec((rows, 2 * MID), lambda b, r: (b * NROWB + r, 0)),
            pl.BlockSpec((rows, 16), lambda b, r: (b * NROWB + r, 0)),
            pl.BlockSpec((None, NBLK, 16), lambda b, r: (b, r, 0)),
            full((16, POS_H)), full((1, POS_H)),
            full((POS_H, MID)), full((1, MID)),
            full((MID, MID * ATTN_M)), full((1, MID * ATTN_M)),
            full((MID * ATTN_M, MID)), full((1, MID)),
            full((MID, HIGH_C)), full((1, HIGH_C)),
        ],
        out_specs=pl.BlockSpec((None, NBLK, HIGH_C), lambda b, r: (b, r, 0)),
        out_shape=jax.ShapeDtypeStruct((B, N, HIGH_C), jnp.float32),
    )(pts, q, kvg, xyzg, xhp,
      w['pw1'], w['pb1'], w['pw2'], w['pb2'],
      w['aw1'], w['ab1'], w['aw2'], w['ab2'], w['afw'], w['afb'])


def kernel(xyz_low, xyz_high, points_low, points_high, params):
    f32 = jnp.float32
    xhT = jnp.transpose(xyz_high, (0, 2, 1))
    plowT = jnp.transpose(points_low, (0, 2, 1))
    phT = jnp.transpose(points_high, (0, 2, 1))

    inv = 1.0 / jnp.sqrt(f32(1.0 + 1e-5))
    scale = inv * params['up_bn_g']
    upwT = (params['up_w'] * scale[:, None]).T
    upb = (params['up_b'] * scale + params['up_bn_b']).reshape(1, HIGH_C)

    points = _interp(plowT, upwT, upb, xhT, xyz_low, phT)

    gidx = _knn(xhT, xyz_high)
    idx_flat = gidx.reshape(B * N * K_NEI)

    xhp = jnp.pad(xhT, ((0, 0), (0, 0), (0, 13)))
    xyz_table = xhp.reshape(B * N, 16)
    xyzg = _sc_gather(xyz_table, idx_flat, 16)

    for blk in params['blocks']:
        w = {
            'pw1': jnp.pad(blk['pos_w1'], ((0, 0), (0, 13))).T,
            'pb1': blk['pos_b1'].reshape(1, POS_H),
            'pw2': blk['pos_w2'].T, 'pb2': blk['pos_b2'].reshape(1, MID),
            'aw1': blk['attn_w1'].T, 'ab1': blk['attn_b1'].reshape(1, MID * ATTN_M),
            'aw2': blk['attn_w2'].T, 'ab2': blk['attn_b2'].reshape(1, MID),
            'afw': blk['after_w'].T, 'afb': blk['after_b'].reshape(1, HIGH_C),
        }
        q, kv = _qkv(points, blk['before_w'].T,
                     blk['before_b'].reshape(1, MID), blk['qkv_w'].T)
        kvg = _sc_gather(kv.reshape(B * N, 2 * MID), idx_flat, 2 * MID)
        points = _attn(points, q, kvg, xyzg, xhp, w)

    return jnp.transpose(points, (0, 2, 1))
